# baseline (speedup 1.0000x reference)
import functools

import numpy as np

import jax
import jax.numpy as jnp
from jax.experimental import pallas as pl
from jax.experimental.pallas import tpu as pltpu

_ROT0 = (13, 15, 26, 6)
_ROT1 = (17, 29, 16, 24)
_KS = (np.uint32(0), np.uint32(42),
       np.uint32(0) ^ np.uint32(42) ^ np.uint32(0x1BD11BDA))
_THRESHOLD = np.uint32(1258292)
_MASK_TOKEN = np.int32(2)

_LANES = 128
_ROWS_PER_BLOCK = 1600
_CHUNK = 200


def _threefry_bits(x1):
    x1 = x1 + _KS[1]
    x0 = x1
    x1 = ((x1 << np.uint32(13)) | (x1 >> np.uint32(19))) ^ x0
    for r in _ROT0[1:]:
        x0 = x0 + x1
        x1 = (x1 << np.uint32(r)) | (x1 >> np.uint32(32 - r))
        x1 = x1 ^ x0
    x0 = x0 + _KS[1]
    x1 = x1 + _KS[2] + np.uint32(1)
    for i in range(1, 5):
        for r in (_ROT0 if i % 2 == 0 else _ROT1):
            x0 = x0 + x1
            x1 = (x1 << np.uint32(r)) | (x1 >> np.uint32(32 - r))
            x1 = x1 ^ x0
        x0 = x0 + _KS[(i + 1) % 3]
        x1 = x1 + _KS[(i + 2) % 3] + np.uint32(i + 1)
    return x0 ^ x1


def _corrupt_block(seq_ref, out_ref, attn_out_ref, *, rows, chunk):
    g = pl.program_id(0)
    base = g * (rows * _LANES)
    r = jax.lax.broadcasted_iota(jnp.uint32, (chunk, _LANES), 0)
    c = jax.lax.broadcasted_iota(jnp.uint32, (chunk, _LANES), 1)
    rel = (r << np.uint32(7)) | c
    ones = jnp.ones((chunk, _LANES), jnp.float32)

    def step(i, carry):
        r0 = i * chunk
        flat = (base + r0 * _LANES).astype(jnp.uint32) + rel
        bits = _threefry_bits(flat)
        corrupt = (bits >> np.uint32(9)) < _THRESHOLD
        out_ref[pl.ds(r0, chunk), :] = jnp.where(
            corrupt, _MASK_TOKEN, seq_ref[pl.ds(r0, chunk), :])
        attn_out_ref[pl.ds(r0, chunk), :] = ones
        return carry

    jax.lax.fori_loop(0, rows // chunk, step, 0)


def kernel(input_sequences, attention_mask):
    batch, seq_len = input_sequences.shape
    total = batch * seq_len
    flat_rows = total // _LANES
    rows = _ROWS_PER_BLOCK
    seq_flat = input_sequences.reshape(flat_rows, _LANES)
    body = functools.partial(_corrupt_block, rows=rows, chunk=_CHUNK)
    spec = pl.BlockSpec((rows, _LANES), lambda g: (g, 0))
    corrupted, attn_out = pl.pallas_call(
        body,
        grid=(flat_rows // rows,),
        in_specs=[spec],
        out_specs=[spec, spec],
        out_shape=[
            jax.ShapeDtypeStruct((flat_rows, _LANES), jnp.int32),
            jax.ShapeDtypeStruct((flat_rows, _LANES), jnp.float32),
        ],
        compiler_params=pltpu.CompilerParams(
            dimension_semantics=("arbitrary",)),
    )(seq_flat)
    return (corrupted.reshape(batch, seq_len),
            attn_out.reshape(batch, seq_len))


# speedup vs baseline: 3.4320x; 3.4320x over previous
import functools

import numpy as np

import jax
import jax.numpy as jnp
from jax.experimental import pallas as pl
from jax.experimental.pallas import tpu as pltpu

_ROT0 = (13, 15, 26, 6)
_ROT1 = (17, 29, 16, 24)
_KS = (np.uint32(0), np.uint32(42),
       np.uint32(0) ^ np.uint32(42) ^ np.uint32(0x1BD11BDA))
_THRESHOLD = np.uint32(1258292)
_MASK_TOKEN = np.int32(2)

_COLS_PER_BLOCK = 512


def _threefry_bits(x1):
    x1 = x1 + _KS[1]
    x0 = x1
    x1 = ((x1 << np.uint32(13)) | (x1 >> np.uint32(19))) ^ x0
    for r in _ROT0[1:]:
        x0 = x0 + x1
        x1 = (x1 << np.uint32(r)) | (x1 >> np.uint32(32 - r))
        x1 = x1 ^ x0
    x0 = x0 + _KS[1]
    x1 = x1 + _KS[2] + np.uint32(1)
    for i in range(1, 5):
        for r in (_ROT0 if i % 2 == 0 else _ROT1):
            x0 = x0 + x1
            x1 = (x1 << np.uint32(r)) | (x1 >> np.uint32(32 - r))
            x1 = x1 ^ x0
        x0 = x0 + _KS[(i + 1) % 3]
        x1 = x1 + _KS[(i + 2) % 3] + np.uint32(i + 1)
    return x0 ^ x1


def _corrupt_block(seq_ref, out_ref, attn_out_ref, *, seq_len, cols):
    g = pl.program_id(0)
    base = (g * (cols * seq_len)).astype(jnp.uint32)
    s = jax.lax.broadcasted_iota(jnp.uint32, (seq_len, cols), 0)
    b = jax.lax.broadcasted_iota(jnp.uint32, (seq_len, cols), 1)
    flat = base + b * np.uint32(seq_len) + s
    bits = _threefry_bits(flat)
    corrupt = (bits >> np.uint32(9)) < _THRESHOLD
    out_ref[...] = jnp.where(corrupt, _MASK_TOKEN, seq_ref[...])
    attn_out_ref[...] = jnp.ones((seq_len, cols), jnp.float32)


def kernel(input_sequences, attention_mask):
    batch, seq_len = input_sequences.shape
    cols = _COLS_PER_BLOCK
    seq_t = input_sequences.T
    body = functools.partial(_corrupt_block, seq_len=seq_len, cols=cols)
    spec = pl.BlockSpec((seq_len, cols), lambda g: (0, g))
    corrupted_t, attn_out_t = pl.pallas_call(
        body,
        grid=(batch // cols,),
        in_specs=[spec],
        out_specs=[spec, spec],
        out_shape=[
            jax.ShapeDtypeStruct((seq_len, batch), jnp.int32),
            jax.ShapeDtypeStruct((seq_len, batch), jnp.float32),
        ],
        compiler_params=pltpu.CompilerParams(
            dimension_semantics=("arbitrary",)),
    )(seq_t)
    return corrupted_t.T, attn_out_t.T


# speedup vs baseline: 3.4427x; 1.0031x over previous
import functools

import numpy as np

import jax
import jax.numpy as jnp
from jax.experimental import pallas as pl
from jax.experimental.pallas import tpu as pltpu

_ROT0 = (13, 15, 26, 6)
_ROT1 = (17, 29, 16, 24)
_KS = (np.uint32(0), np.uint32(42),
       np.uint32(0) ^ np.uint32(42) ^ np.uint32(0x1BD11BDA))
_THRESHOLD = np.uint32(1258292)
_MASK_TOKEN = np.int32(2)

_COLS_PER_BLOCK = 1024


def _threefry_bits(x1):
    x1 = x1 + _KS[1]
    x0 = x1
    x1 = ((x1 << np.uint32(13)) | (x1 >> np.uint32(19))) ^ x0
    for r in _ROT0[1:]:
        x0 = x0 + x1
        x1 = (x1 << np.uint32(r)) | (x1 >> np.uint32(32 - r))
        x1 = x1 ^ x0
    x0 = x0 + _KS[1]
    x1 = x1 + _KS[2] + np.uint32(1)
    for i in range(1, 5):
        for r in (_ROT0 if i % 2 == 0 else _ROT1):
            x0 = x0 + x1
            x1 = (x1 << np.uint32(r)) | (x1 >> np.uint32(32 - r))
            x1 = x1 ^ x0
        x0 = x0 + _KS[(i + 1) % 3]
        x1 = x1 + _KS[(i + 2) % 3] + np.uint32(i + 1)
    return x0 ^ x1


def _corrupt_block(seq_ref, out_ref, attn_out_ref, *, seq_len, cols):
    g = pl.program_id(0)
    base = (g * (cols * seq_len)).astype(jnp.uint32)
    s = jax.lax.broadcasted_iota(jnp.uint32, (seq_len, cols), 0)
    b = jax.lax.broadcasted_iota(jnp.uint32, (seq_len, cols), 1)
    flat = base + b * np.uint32(seq_len) + s
    bits = _threefry_bits(flat)
    corrupt = (bits >> np.uint32(9)) < _THRESHOLD
    out_ref[...] = jnp.where(corrupt, _MASK_TOKEN, seq_ref[...])
    attn_out_ref[...] = jnp.ones((seq_len, cols), jnp.float32)


def kernel(input_sequences, attention_mask):
    batch, seq_len = input_sequences.shape
    cols = _COLS_PER_BLOCK
    seq_t = input_sequences.T
    body = functools.partial(_corrupt_block, seq_len=seq_len, cols=cols)
    spec = pl.BlockSpec((seq_len, cols), lambda g: (0, g))
    corrupted_t, attn_out_t = pl.pallas_call(
        body,
        grid=(batch // cols,),
        in_specs=[spec],
        out_specs=[spec, spec],
        out_shape=[
            jax.ShapeDtypeStruct((seq_len, batch), jnp.int32),
            jax.ShapeDtypeStruct((seq_len, batch), jnp.float32),
        ],
        compiler_params=pltpu.CompilerParams(
            dimension_semantics=("arbitrary",)),
    )(seq_t)
    return corrupted_t.T, attn_out_t.T
